# Initial kernel scaffold; baseline (speedup 1.0000x reference)
import jax
import jax.numpy as jnp
from jax.experimental import pallas as pl


def kernel(h, pos, edge_attr, W_e1, b_e1, W_e2, b_e2, W_x1, b_x1, W_x2, b_x2, edge_index):
    raise NotImplementedError("write your pallas kernel here")


# speedup vs baseline: 2.0176x; 2.0176x over previous
import functools

import jax
import jax.numpy as jnp
from jax import lax
from jax.experimental import pallas as pl
from jax.experimental.pallas import tpu as pltpu
from jax.experimental.pallas import tpu_sc as plsc

NC = 2
NS = 16
NW = NC * NS
CH = 128


def _precompute_ab(h, W_e1, b_e1_row):
    N, D = h.shape

    def body(h_ref, w_ref, b_ref, a_ref, bb_ref):
        hb = h_ref[...]
        wa = w_ref[0:D, :]
        wb = w_ref[D:2 * D, :]
        a_ref[...] = (
            jnp.dot(hb, wa, preferred_element_type=jnp.float32,
                    precision=lax.Precision.HIGHEST) + b_ref[...]
        )
        bb_ref[...] = jnp.dot(hb, wb, preferred_element_type=jnp.float32,
                              precision=lax.Precision.HIGHEST)

    BLK = 2000
    grid = (N // BLK,)
    return pl.pallas_call(
        body,
        grid=grid,
        in_specs=[
            pl.BlockSpec((BLK, D), lambda i: (i, 0)),
            pl.BlockSpec(W_e1.shape, lambda i: (0, 0)),
            pl.BlockSpec((1, D), lambda i: (0, 0)),
        ],
        out_specs=[pl.BlockSpec((BLK, D), lambda i: (i, 0))] * 2,
        out_shape=[jax.ShapeDtypeStruct((N, D), jnp.float32)] * 2,
    )(h, W_e1, b_e1_row)


def _sc_gather(edge_index, A, B, pos16):
    E = edge_index.shape[1]
    N, D = A.shape
    NCHUNK = E // CH
    mesh = plsc.VectorSubcoreMesh(core_axis_name="c", subcore_axis_name="s",
                                  num_cores=NC, num_subcores=NS)

    @functools.partial(
        pl.kernel,
        out_type=[
            jax.ShapeDtypeStruct((E, D), jnp.float32),
            jax.ShapeDtypeStruct((E, D), jnp.float32),
            jax.ShapeDtypeStruct((E, 16), jnp.float32),
            jax.ShapeDtypeStruct((E, 16), jnp.float32),
        ],
        mesh=mesh,
        scratch_types=[
            pltpu.VMEM((CH,), jnp.int32),
            pltpu.VMEM((CH,), jnp.int32),
            pltpu.VMEM((CH, D), jnp.float32),
            pltpu.VMEM((CH, D), jnp.float32),
            pltpu.VMEM((CH, 16), jnp.float32),
            pltpu.VMEM((CH, 16), jnp.float32),
            pltpu.SemaphoreType.DMA,
            pltpu.SemaphoreType.DMA,
            pltpu.SemaphoreType.DMA,
            pltpu.SemaphoreType.DMA,
        ],
        compiler_params=pltpu.CompilerParams(use_tc_tiling_on_sc=False),
    )
    def k(ei, a_hbm, b_hbm, p_hbm, arow_o, bcol_o, pr_o, pc_o,
          rowv, colv, abuf, bbuf, prbuf, pcbuf, s1, s2, s3, s4):
        c = lax.axis_index("c")
        s = lax.axis_index("s")
        wid = s * NC + c
        n_mine = (NCHUNK - wid + NW - 1) // NW

        def body(i, carry):
            base = (wid + i * NW) * CH
            pltpu.sync_copy(ei.at[0, pl.ds(base, CH)], rowv)
            pltpu.sync_copy(ei.at[1, pl.ds(base, CH)], colv)
            ca = pltpu.async_copy(a_hbm.at[rowv], abuf, s1)
            cb = pltpu.async_copy(b_hbm.at[colv], bbuf, s2)
            cp = pltpu.async_copy(p_hbm.at[rowv], prbuf, s3)
            cq = pltpu.async_copy(p_hbm.at[colv], pcbuf, s4)
            ca.wait()
            cb.wait()
            cp.wait()
            cq.wait()
            pltpu.sync_copy(abuf, arow_o.at[pl.ds(base, CH)])
            pltpu.sync_copy(bbuf, bcol_o.at[pl.ds(base, CH)])
            pltpu.sync_copy(prbuf, pr_o.at[pl.ds(base, CH)])
            pltpu.sync_copy(pcbuf, pc_o.at[pl.ds(base, CH)])
            return carry

        lax.fori_loop(0, n_mine, body, 0)

    return k(edge_index, A, B, pos16)


def _tc_edge_mlp(arow, bcol, pr8, pc8, wd_row, W_e2, b_e2_row,
                 W_x1, b_x1_row, W_x2, b_x2_row):
    E, D = arow.shape
    BE = 512

    def body(a_ref, b_ref, pr_ref, pc_ref, wd_ref, w2_ref, b2_ref,
             wx1_ref, bx1_ref, wx2_ref, bx2_ref, msg_ref, posv_ref):
        rel = pr_ref[...] - pc_ref[...]
        dist = jnp.sqrt(jnp.sum(rel * rel, axis=1, keepdims=True))
        pre = a_ref[...] + b_ref[...] + dist * wd_ref[...]
        t = pre * jax.nn.sigmoid(pre)
        msg = jnp.dot(t, w2_ref[...], preferred_element_type=jnp.float32,
                      precision=lax.Precision.HIGHEST) + b2_ref[...]
        msg_ref[...] = msg
        u = jnp.dot(msg, wx1_ref[...], preferred_element_type=jnp.float32,
                    precision=lax.Precision.HIGHEST) + bx1_ref[...]
        u = u * jax.nn.sigmoid(u)
        coeff = jnp.dot(u, wx2_ref[...], preferred_element_type=jnp.float32,
                        precision=lax.Precision.HIGHEST) + bx2_ref[...]
        posv_ref[...] = rel * coeff

    grid = (E // BE,)
    const = lambda shape: pl.BlockSpec(shape, lambda i: tuple(0 for _ in shape))
    return pl.pallas_call(
        body,
        grid=grid,
        in_specs=[
            pl.BlockSpec((BE, D), lambda i: (i, 0)),
            pl.BlockSpec((BE, D), lambda i: (i, 0)),
            pl.BlockSpec((BE, 16), lambda i: (i, 0)),
            pl.BlockSpec((BE, 16), lambda i: (i, 0)),
            const((1, D)),
            const((D, D)),
            const((1, D)),
            const((D, D)),
            const((1, D)),
            const((D, 1)),
            const((1, 1)),
        ],
        out_specs=[
            pl.BlockSpec((BE, D), lambda i: (i, 0)),
            pl.BlockSpec((BE, 16), lambda i: (i, 0)),
        ],
        out_shape=[
            jax.ShapeDtypeStruct((E, D), jnp.float32),
            jax.ShapeDtypeStruct((E, 16), jnp.float32),
        ],
    )(arow, bcol, pr8, pc8, wd_row, W_e2, b_e2_row,
      W_x1, b_x1_row, W_x2, b_x2_row)


def _sc_scatter(edge_index, h, msg, posv):
    E = edge_index.shape[1]
    N, D = h.shape
    NCHUNK = E // CH
    ROWS_PER_TILE = N // NS
    mesh = plsc.VectorSubcoreMesh(core_axis_name="c", subcore_axis_name="s",
                                  num_cores=NC, num_subcores=NS)

    @functools.partial(
        pl.kernel,
        out_type=[
            jax.ShapeDtypeStruct((NC, N, D), jnp.float32),
            jax.ShapeDtypeStruct((NC, N, 16), jnp.float32),
        ],
        mesh=mesh,
        scratch_types=[
            pltpu.VMEM((CH,), jnp.int32),
            pltpu.VMEM((CH,), jnp.int32),
            pltpu.VMEM((CH, D), jnp.float32),
            pltpu.VMEM((CH, D), jnp.float32),
            pltpu.VMEM((CH, 16), jnp.float32),
            pltpu.SemaphoreType.DMA,
            pltpu.VMEM_SHARED((N, D), jnp.float32),
            pltpu.VMEM_SHARED((N, 16), jnp.float32),
        ],
        compiler_params=pltpu.CompilerParams(use_tc_tiling_on_sc=False),
    )
    def k(ei, h_hbm, m_hbm, pv_hbm, hp_o, pp_o,
          rowv, colv, hbuf, mbuf, pbuf, s1, hsh, psh):
        c = lax.axis_index("c")
        s = lax.axis_index("s")
        wid = s * NC + c
        n_mine = (NCHUNK - wid + NW - 1) // NW

        if True:
            def zero_hbuf(kk, carry):
                i = kk // 8
                j = (kk % 8) * 16
                hbuf[i, pl.ds(j, 16)] = jnp.zeros((16,), jnp.float32)
                return carry

            lax.fori_loop(0, CH * 8, zero_hbuf, 0)

            def zero_pbuf(kk, carry):
                pbuf[kk, pl.ds(0, 16)] = jnp.zeros((16,), jnp.float32)
                return carry

            lax.fori_loop(0, CH, zero_pbuf, 0)

            r0 = s * ROWS_PER_TILE
            full = ROWS_PER_TILE // CH
            tail = ROWS_PER_TILE - full * CH

            def zero_sh(q, carry):
                pltpu.sync_copy(hbuf, hsh.at[pl.ds(r0 + q * CH, CH)])
                pltpu.sync_copy(pbuf, psh.at[pl.ds(r0 + q * CH, CH)])
                return carry

            lax.fori_loop(0, full, zero_sh, 0)
            pltpu.sync_copy(hbuf.at[pl.ds(0, tail)],
                            hsh.at[pl.ds(r0 + full * CH, tail)])
            pltpu.sync_copy(pbuf.at[pl.ds(0, tail)],
                            psh.at[pl.ds(r0 + full * CH, tail)])
            plsc.subcore_barrier()

            def body(i, carry):
                base = (wid + i * NW) * CH
                pltpu.sync_copy(ei.at[0, pl.ds(base, CH)], rowv)
                pltpu.sync_copy(ei.at[1, pl.ds(base, CH)], colv)
                ch = pltpu.async_copy(h_hbm.at[rowv], hbuf, s1)
                pltpu.sync_copy(m_hbm.at[pl.ds(base, CH)], mbuf)
                pltpu.sync_copy(pv_hbm.at[pl.ds(base, CH)], pbuf)
                ch.wait()

                def mul(kk, carry2):
                    ii = kk // 8
                    jj = (kk % 8) * 16
                    hbuf[ii, pl.ds(jj, 16)] = (
                        hbuf[ii, pl.ds(jj, 16)] * mbuf[ii, pl.ds(jj, 16)]
                    )
                    return carry2

                lax.fori_loop(0, CH * 8, mul, 0)
                pltpu.sync_copy(hbuf, hsh.at[colv], add=True)
                pltpu.sync_copy(pbuf, psh.at[colv], add=True)
                return carry

            lax.fori_loop(0, n_mine, body, 0)
            plsc.subcore_barrier()
            pltpu.sync_copy(hsh.at[pl.ds(r0, ROWS_PER_TILE)],
                            hp_o.at[c, pl.ds(r0, ROWS_PER_TILE)])
            pltpu.sync_copy(psh.at[pl.ds(r0, ROWS_PER_TILE)],
                            pp_o.at[c, pl.ds(r0, ROWS_PER_TILE)])


    return k(edge_index, h, msg, posv)


def _combine_h(hp):
    _, N, D = hp.shape
    BLK = 2000

    def body(hp_ref, out_ref):
        out_ref[...] = hp_ref[0] + hp_ref[1]

    return pl.pallas_call(
        body,
        grid=(N // BLK,),
        in_specs=[pl.BlockSpec((NC, BLK, D), lambda i: (0, i, 0))],
        out_specs=pl.BlockSpec((BLK, D), lambda i: (i, 0)),
        out_shape=jax.ShapeDtypeStruct((N, D), jnp.float32),
    )(hp)


def _combine_pos(pp, num_out):
    _, N, _ = pp.shape

    def body(pp_ref, out_ref):
        i = pl.program_id(0)
        val = pp_ref[0, :, 0:3] + pp_ref[1, :, 0:3]
        out_ref[...] = jnp.where(i == 0, val, jnp.zeros_like(val))

    return pl.pallas_call(
        body,
        grid=(num_out // N,),
        in_specs=[pl.BlockSpec((NC, N, 16), lambda i: (0, 0, 0))],
        out_specs=pl.BlockSpec((N, 3), lambda i: (i, 0)),
        out_shape=jax.ShapeDtypeStruct((num_out, 3), jnp.float32),
    )(pp)


def kernel(h, pos, edge_attr, W_e1, b_e1, W_e2, b_e2,
           W_x1, b_x1, W_x2, b_x2, edge_index):
    del edge_attr
    N, D = h.shape
    E = edge_index.shape[1]
    num_out = max(N, E)

    pos16 = jnp.pad(pos, ((0, 0), (0, 13)))
    wd_row = W_e1[2 * D].reshape(1, D)

    A, B = _precompute_ab(h, W_e1, b_e1.reshape(1, D))
    arow, bcol, pr8, pc8 = _sc_gather(edge_index, A, B, pos16)
    msg, posv = _tc_edge_mlp(arow, bcol, pr8, pc8, wd_row, W_e2,
                             b_e2.reshape(1, D), W_x1, b_x1.reshape(1, D),
                             W_x2, b_x2.reshape(1, 1))
    hp, pp = _sc_scatter(edge_index, h, msg, posv)
    h_msg = _combine_h(hp)
    pos_msg = _combine_pos(pp, num_out)
    return (h_msg, pos_msg)


# speedup vs baseline: 2.1891x; 1.0850x over previous
import functools

import jax
import jax.numpy as jnp
from jax import lax
from jax.experimental import pallas as pl
from jax.experimental.pallas import tpu as pltpu
from jax.experimental.pallas import tpu_sc as plsc

NC = 2
NS = 16
NW = NC * NS
CH = 128


def _dot3(x, w):
    xh = x.astype(jnp.bfloat16)
    xl = (x - xh.astype(jnp.float32)).astype(jnp.bfloat16)
    wh = w.astype(jnp.bfloat16)
    wl = (w - wh.astype(jnp.float32)).astype(jnp.bfloat16)
    acc = jnp.dot(xh, wh, preferred_element_type=jnp.float32)
    acc += jnp.dot(xl, wh, preferred_element_type=jnp.float32)
    acc += jnp.dot(xh, wl, preferred_element_type=jnp.float32)
    return acc


def _precompute_ab(h, W_e1, b_e1_row):
    N, D = h.shape

    def body(h_ref, w_ref, b_ref, a_ref, bb_ref):
        hb = h_ref[...]
        wa = w_ref[0:D, :]
        wb = w_ref[D:2 * D, :]
        a_ref[...] = (
            jnp.dot(hb, wa, preferred_element_type=jnp.float32,
                    precision=lax.Precision.HIGHEST) + b_ref[...]
        )
        bb_ref[...] = jnp.dot(hb, wb, preferred_element_type=jnp.float32,
                              precision=lax.Precision.HIGHEST)

    BLK = 2000
    grid = (N // BLK,)
    return pl.pallas_call(
        body,
        grid=grid,
        in_specs=[
            pl.BlockSpec((BLK, D), lambda i: (i, 0)),
            pl.BlockSpec(W_e1.shape, lambda i: (0, 0)),
            pl.BlockSpec((1, D), lambda i: (0, 0)),
        ],
        out_specs=[pl.BlockSpec((BLK, D), lambda i: (i, 0))] * 2,
        out_shape=[jax.ShapeDtypeStruct((N, D), jnp.float32)] * 2,
    )(h, W_e1, b_e1_row)


def _sc_gather(edge_index, A, B, pos16):
    E = edge_index.shape[1]
    N, D = A.shape
    NCHUNK = E // CH
    mesh = plsc.VectorSubcoreMesh(core_axis_name="c", subcore_axis_name="s",
                                  num_cores=NC, num_subcores=NS)

    @functools.partial(
        pl.kernel,
        out_type=[
            jax.ShapeDtypeStruct((E, D), jnp.float32),
            jax.ShapeDtypeStruct((E, 16), jnp.float32),
        ],
        mesh=mesh,
        scratch_types=[
            pltpu.VMEM((CH,), jnp.int32), pltpu.VMEM((CH,), jnp.int32),
            pltpu.VMEM((CH,), jnp.int32), pltpu.VMEM((CH,), jnp.int32),
            pltpu.VMEM((CH, D), jnp.float32), pltpu.VMEM((CH, D), jnp.float32),
            pltpu.VMEM((CH, D), jnp.float32), pltpu.VMEM((CH, D), jnp.float32),
            pltpu.VMEM((CH, 16), jnp.float32), pltpu.VMEM((CH, 16), jnp.float32),
            pltpu.VMEM((CH, 16), jnp.float32), pltpu.VMEM((CH, 16), jnp.float32),
            pltpu.SemaphoreType.DMA, pltpu.SemaphoreType.DMA,
            pltpu.SemaphoreType.DMA, pltpu.SemaphoreType.DMA,
        ],
        compiler_params=pltpu.CompilerParams(use_tc_tiling_on_sc=False),
    )
    def k(ei, a_hbm, b_hbm, p_hbm, ps_o, rel_o,
          row0, col0, row1, col1, ab0, ab1, bb0, bb1, pr0, pr1, pc0, pc1,
          g0, g1, w0, w1):
        c = lax.axis_index("c")
        s = lax.axis_index("s")
        wid = s * NC + c
        n_mine = (NCHUNK - wid + NW - 1) // NW
        slots = ((row0, col0, ab0, bb0, pr0, pc0, g0, w0),
                 (row1, col1, ab1, bb1, pr1, pc1, g1, w1))

        def chunk_base(i):
            return (wid + i * NW) * CH

        def issue(i, slot):
            rowv, colv, abuf, bbuf, prbuf, pcbuf, g, w = slot
            base = chunk_base(i)
            pltpu.sync_copy(ei.at[0, pl.ds(base, CH)], rowv)
            pltpu.sync_copy(ei.at[1, pl.ds(base, CH)], colv)
            pltpu.async_copy(a_hbm.at[rowv], abuf, g)
            pltpu.async_copy(b_hbm.at[colv], bbuf, g)
            pltpu.async_copy(p_hbm.at[rowv], prbuf, g)
            pltpu.async_copy(p_hbm.at[colv], pcbuf, g)

        def wait_gathers(slot):
            rowv, colv, abuf, bbuf, prbuf, pcbuf, g, w = slot
            pltpu.make_async_copy(a_hbm.at[rowv], abuf, g).wait()
            pltpu.make_async_copy(b_hbm.at[colv], bbuf, g).wait()
            pltpu.make_async_copy(p_hbm.at[rowv], prbuf, g).wait()
            pltpu.make_async_copy(p_hbm.at[colv], pcbuf, g).wait()

        def compute_and_wb(i, slot):
            rowv, colv, abuf, bbuf, prbuf, pcbuf, g, w = slot

            def add(kk, carry):
                ii = kk // 8
                jj = (kk % 8) * 16
                abuf[ii, pl.ds(jj, 16)] = (
                    abuf[ii, pl.ds(jj, 16)] + bbuf[ii, pl.ds(jj, 16)]
                )
                return carry

            lax.fori_loop(0, CH * 8, add, 0)

            def sub(kk, carry):
                prbuf[kk, pl.ds(0, 16)] = (
                    prbuf[kk, pl.ds(0, 16)] - pcbuf[kk, pl.ds(0, 16)]
                )
                return carry

            lax.fori_loop(0, CH, sub, 0)
            base = chunk_base(i)
            pltpu.async_copy(abuf, ps_o.at[pl.ds(base, CH)], w)
            pltpu.async_copy(prbuf, rel_o.at[pl.ds(base, CH)], w)

        def wait_wb(slot):
            rowv, colv, abuf, bbuf, prbuf, pcbuf, g, w = slot
            pltpu.make_async_copy(abuf, ps_o.at[pl.ds(0, CH)], w).wait()
            pltpu.make_async_copy(prbuf, rel_o.at[pl.ds(0, CH)], w).wait()

        @pl.when(n_mine > 0)
        def _():
            issue(0, slots[0])

        def body(i2, carry):
            a = 2 * i2
            b = a + 1

            @pl.when(b < n_mine)
            def _():
                @pl.when(i2 > 0)
                def _():
                    wait_wb(slots[1])
                issue(b, slots[1])

            wait_gathers(slots[0])
            compute_and_wb(a, slots[0])

            @pl.when(b < n_mine)
            def _():
                wait_gathers(slots[1])
                compute_and_wb(b, slots[1])

            @pl.when(a + 2 < n_mine)
            def _():
                wait_wb(slots[0])
                issue(a + 2, slots[0])

            return carry

        lax.fori_loop(0, (n_mine + 1) // 2, body, 0)

        @pl.when(n_mine > 0)
        def _():
            wait_wb(slots[0])

        @pl.when(n_mine > 1)
        def _():
            wait_wb(slots[1])

    return k(edge_index, A, B, pos16)


def _tc_edge_mlp(presum, rel16, wd_row, W_e2, b_e2_row,
                 W_x1, b_x1_row, W_x2, b_x2_row):
    E, D = presum.shape
    BE = 1280

    def body(ps_ref, rel_ref, wd_ref, w2_ref, b2_ref,
             wx1_ref, bx1_ref, wx2_ref, bx2_ref, msgl_ref, msgr_ref, posv_ref):
        rel = rel_ref[...]
        dist = jnp.sqrt(jnp.sum(rel * rel, axis=1, keepdims=True))
        pre = ps_ref[...] + dist * wd_ref[...]
        t = pre * jax.nn.sigmoid(pre)
        msg = _dot3(t, w2_ref[...]) + b2_ref[...]
        msgl_ref[...] = msg[:, 0:64]
        msgr_ref[...] = msg[:, 64:128]
        u = _dot3(msg, wx1_ref[...]) + bx1_ref[...]
        u = u * jax.nn.sigmoid(u)
        coeff = jnp.dot(u, wx2_ref[...], preferred_element_type=jnp.float32,
                        precision=lax.Precision.HIGHEST) + bx2_ref[...]
        posv_ref[...] = rel * coeff

    grid = (E // BE,)
    const = lambda shape: pl.BlockSpec(shape, lambda i: tuple(0 for _ in shape))
    return pl.pallas_call(
        body,
        grid=grid,
        in_specs=[
            pl.BlockSpec((BE, D), lambda i: (i, 0)),
            pl.BlockSpec((BE, 16), lambda i: (i, 0)),
            const((1, D)),
            const((D, D)),
            const((1, D)),
            const((D, D)),
            const((1, D)),
            const((D, 1)),
            const((1, 1)),
        ],
        out_specs=[
            pl.BlockSpec((BE, 64), lambda i: (i, 0)),
            pl.BlockSpec((BE, 64), lambda i: (i, 0)),
            pl.BlockSpec((BE, 16), lambda i: (i, 0)),
        ],
        out_shape=[
            jax.ShapeDtypeStruct((E, 64), jnp.float32),
            jax.ShapeDtypeStruct((E, 64), jnp.float32),
            jax.ShapeDtypeStruct((E, 16), jnp.float32),
        ],
    )(presum, rel16, wd_row, W_e2, b_e2_row,
      W_x1, b_x1_row, W_x2, b_x2_row)


def _sc_scatter(edge_index, hL, hR, mL, mR, posv):
    E = edge_index.shape[1]
    N = hL.shape[0]
    DH = hL.shape[1]
    NCHUNK = E // CH
    ROWS_PER_TILE = N // NS
    mesh = plsc.VectorSubcoreMesh(core_axis_name="c", subcore_axis_name="s",
                                  num_cores=NC, num_subcores=NS)

    @functools.partial(
        pl.kernel,
        out_type=[
            jax.ShapeDtypeStruct((NC, N, DH), jnp.float32),
            jax.ShapeDtypeStruct((N, 16), jnp.float32),
        ],
        mesh=mesh,
        scratch_types=[
            pltpu.VMEM((CH,), jnp.int32), pltpu.VMEM((CH,), jnp.int32),
            pltpu.VMEM((CH,), jnp.int32), pltpu.VMEM((CH,), jnp.int32),
            pltpu.VMEM((CH, DH), jnp.float32), pltpu.VMEM((CH, DH), jnp.float32),
            pltpu.VMEM((CH, DH), jnp.float32), pltpu.VMEM((CH, DH), jnp.float32),
            pltpu.VMEM((CH, 16), jnp.float32), pltpu.VMEM((CH, 16), jnp.float32),
            pltpu.SemaphoreType.DMA, pltpu.SemaphoreType.DMA,
            pltpu.SemaphoreType.DMA, pltpu.SemaphoreType.DMA,
            pltpu.VMEM_SHARED((N, DH), jnp.float32),
            pltpu.VMEM_SHARED((N, 16), jnp.float32),
        ],
        compiler_params=pltpu.CompilerParams(use_tc_tiling_on_sc=False),
    )
    def k(ei, hl_hbm, hr_hbm, ml_hbm, mr_hbm, pv_hbm, hp_o, pp_o,
          row0, col0, row1, col1, hb0, hb1, mb0, mb1, pb0, pb1,
          g0, g1, w0, w1, hsh, psh):
        c = lax.axis_index("c")
        s = lax.axis_index("s")
        n_mine = (NCHUNK - s + NS - 1) // NS
        slots = ((row0, col0, hb0, mb0, pb0, g0, w0),
                 (row1, col1, hb1, mb1, pb1, g1, w1))

        def zero_hbuf(kk, carry):
            i = kk // 4
            j = (kk % 4) * 16
            hb0[i, pl.ds(j, 16)] = jnp.zeros((16,), jnp.float32)
            return carry

        lax.fori_loop(0, CH * 4, zero_hbuf, 0)

        def zero_pbuf(kk, carry):
            pb0[kk, pl.ds(0, 16)] = jnp.zeros((16,), jnp.float32)
            return carry

        lax.fori_loop(0, CH, zero_pbuf, 0)

        r0 = s * ROWS_PER_TILE
        full = ROWS_PER_TILE // CH
        tail = ROWS_PER_TILE - full * CH

        def zero_sh(q, carry):
            pltpu.sync_copy(hb0, hsh.at[pl.ds(r0 + q * CH, CH)])
            pltpu.sync_copy(pb0, psh.at[pl.ds(r0 + q * CH, CH)])
            return carry

        lax.fori_loop(0, full, zero_sh, 0)

        @pl.when(tail > 0)
        def _():
            pltpu.sync_copy(hb0.at[pl.ds(0, tail)],
                            hsh.at[pl.ds(r0 + full * CH, tail)])
            pltpu.sync_copy(pb0.at[pl.ds(0, tail)],
                            psh.at[pl.ds(r0 + full * CH, tail)])

        plsc.subcore_barrier()

        def chunk_base(i):
            return (s + i * NS) * CH

        def issue(i, slot):
            rowv, colv, hbuf, mbuf, pbuf, g, w = slot
            base = chunk_base(i)
            pltpu.sync_copy(ei.at[0, pl.ds(base, CH)], rowv)
            pltpu.sync_copy(ei.at[1, pl.ds(base, CH)], colv)

            @pl.when(c == 0)
            def _():
                pltpu.async_copy(hl_hbm.at[rowv], hbuf, g)
                pltpu.async_copy(ml_hbm.at[pl.ds(base, CH)], mbuf, g)
                pltpu.async_copy(pv_hbm.at[pl.ds(base, CH)], pbuf, g)

            @pl.when(c == 1)
            def _():
                pltpu.async_copy(hr_hbm.at[rowv], hbuf, g)
                pltpu.async_copy(mr_hbm.at[pl.ds(base, CH)], mbuf, g)

        def wait_gathers(i, slot):
            rowv, colv, hbuf, mbuf, pbuf, g, w = slot
            base = chunk_base(i)
            pltpu.make_async_copy(hl_hbm.at[rowv], hbuf, g).wait()
            pltpu.make_async_copy(ml_hbm.at[pl.ds(base, CH)], mbuf, g).wait()

            @pl.when(c == 0)
            def _():
                pltpu.make_async_copy(pv_hbm.at[pl.ds(base, CH)], pbuf, g).wait()

        def compute_and_scatter(slot):
            rowv, colv, hbuf, mbuf, pbuf, g, w = slot

            def mul(kk, carry):
                ii = kk // 4
                jj = (kk % 4) * 16
                hbuf[ii, pl.ds(jj, 16)] = (
                    hbuf[ii, pl.ds(jj, 16)] * mbuf[ii, pl.ds(jj, 16)]
                )
                return carry

            lax.fori_loop(0, CH * 4, mul, 0)
            pltpu.async_copy(hbuf, hsh.at[colv], w, add=True)

            @pl.when(c == 0)
            def _():
                pltpu.async_copy(pbuf, psh.at[colv], w, add=True)

        def wait_scatter(slot):
            rowv, colv, hbuf, mbuf, pbuf, g, w = slot
            pltpu.make_async_copy(hbuf, hsh.at[colv], w).wait()

            @pl.when(c == 0)
            def _():
                pltpu.make_async_copy(pbuf, psh.at[colv], w).wait()

        @pl.when(n_mine > 0)
        def _():
            issue(0, slots[0])

        def body(i2, carry):
            a = 2 * i2
            b = a + 1

            @pl.when(b < n_mine)
            def _():
                @pl.when(i2 > 0)
                def _():
                    wait_scatter(slots[1])
                issue(b, slots[1])

            wait_gathers(a, slots[0])
            compute_and_scatter(slots[0])

            @pl.when(b < n_mine)
            def _():
                wait_gathers(b, slots[1])
                compute_and_scatter(slots[1])

            @pl.when(a + 2 < n_mine)
            def _():
                wait_scatter(slots[0])
                issue(a + 2, slots[0])

            return carry

        lax.fori_loop(0, (n_mine + 1) // 2, body, 0)

        @pl.when(n_mine > 0)
        def _():
            wait_scatter(slots[0])

        @pl.when(n_mine > 1)
        def _():
            wait_scatter(slots[1])

        plsc.subcore_barrier()
        pltpu.sync_copy(hsh.at[pl.ds(r0, ROWS_PER_TILE)],
                        hp_o.at[c, pl.ds(r0, ROWS_PER_TILE)])

        @pl.when(c == 0)
        def _():
            pltpu.sync_copy(psh.at[pl.ds(r0, ROWS_PER_TILE)],
                            pp_o.at[pl.ds(r0, ROWS_PER_TILE)])

    return k(edge_index, hL, hR, mL, mR, posv)


def _assemble_h(hp):
    _, N, DH = hp.shape
    BLK = 2000

    def body(hp_ref, out_ref):
        out_ref[:, 0:DH] = hp_ref[0]
        out_ref[:, DH:2 * DH] = hp_ref[1]

    return pl.pallas_call(
        body,
        grid=(N // BLK,),
        in_specs=[pl.BlockSpec((NC, BLK, DH), lambda i: (0, i, 0))],
        out_specs=pl.BlockSpec((BLK, 2 * DH), lambda i: (i, 0)),
        out_shape=jax.ShapeDtypeStruct((N, 2 * DH), jnp.float32),
    )(hp)


def _expand_pos(pp, num_out):
    N, _ = pp.shape

    def body(pp_ref, out_ref):
        i = pl.program_id(0)
        val = pp_ref[:, 0:3]
        out_ref[...] = jnp.where(i == 0, val, jnp.zeros_like(val))

    return pl.pallas_call(
        body,
        grid=(num_out // N,),
        in_specs=[pl.BlockSpec((N, 16), lambda i: (0, 0))],
        out_specs=pl.BlockSpec((N, 3), lambda i: (i, 0)),
        out_shape=jax.ShapeDtypeStruct((num_out, 3), jnp.float32),
    )(pp)


def kernel(h, pos, edge_attr, W_e1, b_e1, W_e2, b_e2,
           W_x1, b_x1, W_x2, b_x2, edge_index):
    del edge_attr
    N, D = h.shape
    E = edge_index.shape[1]
    num_out = max(N, E)

    pos16 = jnp.pad(pos, ((0, 0), (0, 13)))
    wd_row = W_e1[2 * D].reshape(1, D)
    hL = h[:, 0:64]
    hR = h[:, 64:128]

    A, B = _precompute_ab(h, W_e1, b_e1.reshape(1, D))
    presum, rel16 = _sc_gather(edge_index, A, B, pos16)
    mL, mR, posv = _tc_edge_mlp(presum, rel16, wd_row, W_e2,
                                b_e2.reshape(1, D), W_x1, b_x1.reshape(1, D),
                                W_x2, b_x2.reshape(1, 1))
    hp, pp = _sc_scatter(edge_index, hL, hR, mL, mR, posv)
    h_msg = _assemble_h(hp)
    pos_msg = _expand_pos(pp, num_out)
    return (h_msg, pos_msg)


# speedup vs baseline: 2.8089x; 1.2831x over previous
import functools

import jax
import jax.numpy as jnp
from jax import lax
from jax.experimental import pallas as pl
from jax.experimental.pallas import tpu as pltpu
from jax.experimental.pallas import tpu_sc as plsc

NC = 2
NS = 16
NW = NC * NS
CH = 128


def _dot3(x, w):
    xh = x.astype(jnp.bfloat16)
    xl = (x - xh.astype(jnp.float32)).astype(jnp.bfloat16)
    wh = w.astype(jnp.bfloat16)
    wl = (w - wh.astype(jnp.float32)).astype(jnp.bfloat16)
    acc = jnp.dot(xh, wh, preferred_element_type=jnp.float32)
    acc += jnp.dot(xl, wh, preferred_element_type=jnp.float32)
    acc += jnp.dot(xh, wl, preferred_element_type=jnp.float32)
    return acc


def _precompute_ab(h, W_e1, b_e1_row):
    N, D = h.shape

    def body(h_ref, w_ref, b_ref, a_ref, bb_ref):
        hb = h_ref[...]
        wa = w_ref[0:D, :]
        wb = w_ref[D:2 * D, :]
        a_ref[...] = (
            jnp.dot(hb, wa, preferred_element_type=jnp.float32,
                    precision=lax.Precision.HIGHEST) + b_ref[...]
        )
        bb_ref[...] = jnp.dot(hb, wb, preferred_element_type=jnp.float32,
                              precision=lax.Precision.HIGHEST)

    BLK = 2000
    grid = (N // BLK,)
    return pl.pallas_call(
        body,
        grid=grid,
        in_specs=[
            pl.BlockSpec((BLK, D), lambda i: (i, 0)),
            pl.BlockSpec(W_e1.shape, lambda i: (0, 0)),
            pl.BlockSpec((1, D), lambda i: (0, 0)),
        ],
        out_specs=[pl.BlockSpec((BLK, D), lambda i: (i, 0))] * 2,
        out_shape=[jax.ShapeDtypeStruct((N, D), jnp.float32)] * 2,
    )(h, W_e1, b_e1_row)


def _sc_gather(edge_index, A, B, pos16):
    E = edge_index.shape[1]
    N, D = A.shape
    NCHUNK = E // CH
    mesh = plsc.VectorSubcoreMesh(core_axis_name="c", subcore_axis_name="s",
                                  num_cores=NC, num_subcores=NS)

    @functools.partial(
        pl.kernel,
        out_type=[
            jax.ShapeDtypeStruct((E, D), jnp.float32),
            jax.ShapeDtypeStruct((E, 16), jnp.float32),
        ],
        mesh=mesh,
        scratch_types=[
            pltpu.VMEM((CH,), jnp.int32), pltpu.VMEM((CH,), jnp.int32),
            pltpu.VMEM((CH,), jnp.int32), pltpu.VMEM((CH,), jnp.int32),
            pltpu.VMEM((CH, D), jnp.float32), pltpu.VMEM((CH, D), jnp.float32),
            pltpu.VMEM((CH, D), jnp.float32), pltpu.VMEM((CH, D), jnp.float32),
            pltpu.VMEM((CH, 16), jnp.float32), pltpu.VMEM((CH, 16), jnp.float32),
            pltpu.VMEM((CH, 16), jnp.float32), pltpu.VMEM((CH, 16), jnp.float32),
            pltpu.SemaphoreType.DMA, pltpu.SemaphoreType.DMA,
            pltpu.SemaphoreType.DMA, pltpu.SemaphoreType.DMA,
        ],
        compiler_params=pltpu.CompilerParams(use_tc_tiling_on_sc=False),
    )
    def k(ei, a_hbm, b_hbm, p_hbm, ps_o, rel_o,
          row0, col0, row1, col1, ab0, ab1, bb0, bb1, pr0, pr1, pc0, pc1,
          g0, g1, w0, w1):
        c = lax.axis_index("c")
        s = lax.axis_index("s")
        wid = s * NC + c
        n_mine = (NCHUNK - wid + NW - 1) // NW
        slots = ((row0, col0, ab0, bb0, pr0, pc0, g0, w0),
                 (row1, col1, ab1, bb1, pr1, pc1, g1, w1))

        def chunk_base(i):
            return (wid + i * NW) * CH

        def issue(i, slot):
            rowv, colv, abuf, bbuf, prbuf, pcbuf, g, w = slot
            base = chunk_base(i)
            pltpu.sync_copy(ei.at[0, pl.ds(base, CH)], rowv)
            pltpu.sync_copy(ei.at[1, pl.ds(base, CH)], colv)
            pltpu.async_copy(a_hbm.at[rowv], abuf, g)
            pltpu.async_copy(b_hbm.at[colv], bbuf, g)
            pltpu.async_copy(p_hbm.at[rowv], prbuf, g)
            pltpu.async_copy(p_hbm.at[colv], pcbuf, g)

        def wait_gathers(slot):
            rowv, colv, abuf, bbuf, prbuf, pcbuf, g, w = slot
            pltpu.make_async_copy(a_hbm.at[rowv], abuf, g).wait()
            pltpu.make_async_copy(b_hbm.at[colv], bbuf, g).wait()
            pltpu.make_async_copy(p_hbm.at[rowv], prbuf, g).wait()
            pltpu.make_async_copy(p_hbm.at[colv], pcbuf, g).wait()

        def compute_and_wb(i, slot):
            rowv, colv, abuf, bbuf, prbuf, pcbuf, g, w = slot

            @plsc.parallel_loop(0, CH, unroll=4)
            def add(ii):
                for jj in range(0, D, 16):
                    abuf[ii, pl.ds(jj, 16)] = (
                        abuf[ii, pl.ds(jj, 16)] + bbuf[ii, pl.ds(jj, 16)]
                    )
                prbuf[ii, pl.ds(0, 16)] = (
                    prbuf[ii, pl.ds(0, 16)] - pcbuf[ii, pl.ds(0, 16)]
                )
            base = chunk_base(i)
            pltpu.async_copy(abuf, ps_o.at[pl.ds(base, CH)], w)
            pltpu.async_copy(prbuf, rel_o.at[pl.ds(base, CH)], w)

        def wait_wb(slot):
            rowv, colv, abuf, bbuf, prbuf, pcbuf, g, w = slot
            pltpu.make_async_copy(abuf, ps_o.at[pl.ds(0, CH)], w).wait()
            pltpu.make_async_copy(prbuf, rel_o.at[pl.ds(0, CH)], w).wait()

        @pl.when(n_mine > 0)
        def _():
            issue(0, slots[0])

        def body(i2, carry):
            a = 2 * i2
            b = a + 1

            @pl.when(b < n_mine)
            def _():
                @pl.when(i2 > 0)
                def _():
                    wait_wb(slots[1])
                issue(b, slots[1])

            wait_gathers(slots[0])
            compute_and_wb(a, slots[0])

            @pl.when(b < n_mine)
            def _():
                wait_gathers(slots[1])
                compute_and_wb(b, slots[1])

            @pl.when(a + 2 < n_mine)
            def _():
                wait_wb(slots[0])
                issue(a + 2, slots[0])

            return carry

        lax.fori_loop(0, (n_mine + 1) // 2, body, 0)

        @pl.when(n_mine > 0)
        def _():
            wait_wb(slots[0])

        @pl.when(n_mine > 1)
        def _():
            wait_wb(slots[1])

    return k(edge_index, A, B, pos16)


def _tc_edge_mlp(presum, rel16, wd_row, W_e2, b_e2_row,
                 W_x1, b_x1_row, W_x2, b_x2_row):
    E, D = presum.shape
    BE = 1280

    def body(ps_ref, rel_ref, wd_ref, w2_ref, b2_ref,
             wx1_ref, bx1_ref, wx2_ref, bx2_ref, msgl_ref, msgr_ref, posv_ref):
        rel = rel_ref[...]
        dist = jnp.sqrt(jnp.sum(rel * rel, axis=1, keepdims=True))
        pre = ps_ref[...] + dist * wd_ref[...]
        t = pre * jax.nn.sigmoid(pre)
        msg = _dot3(t, w2_ref[...]) + b2_ref[...]
        msgl_ref[...] = msg[:, 0:64]
        msgr_ref[...] = msg[:, 64:128]
        u = _dot3(msg, wx1_ref[...]) + bx1_ref[...]
        u = u * jax.nn.sigmoid(u)
        coeff = jnp.dot(u, wx2_ref[...], preferred_element_type=jnp.float32,
                        precision=lax.Precision.HIGHEST) + bx2_ref[...]
        posv_ref[...] = rel * coeff

    grid = (E // BE,)
    const = lambda shape: pl.BlockSpec(shape, lambda i: tuple(0 for _ in shape))
    return pl.pallas_call(
        body,
        grid=grid,
        in_specs=[
            pl.BlockSpec((BE, D), lambda i: (i, 0)),
            pl.BlockSpec((BE, 16), lambda i: (i, 0)),
            const((1, D)),
            const((D, D)),
            const((1, D)),
            const((D, D)),
            const((1, D)),
            const((D, 1)),
            const((1, 1)),
        ],
        out_specs=[
            pl.BlockSpec((BE, 64), lambda i: (i, 0)),
            pl.BlockSpec((BE, 64), lambda i: (i, 0)),
            pl.BlockSpec((BE, 16), lambda i: (i, 0)),
        ],
        out_shape=[
            jax.ShapeDtypeStruct((E, 64), jnp.float32),
            jax.ShapeDtypeStruct((E, 64), jnp.float32),
            jax.ShapeDtypeStruct((E, 16), jnp.float32),
        ],
    )(presum, rel16, wd_row, W_e2, b_e2_row,
      W_x1, b_x1_row, W_x2, b_x2_row)


def _sc_scatter(edge_index, hL, hR, mL, mR, posv):
    E = edge_index.shape[1]
    N = hL.shape[0]
    DH = hL.shape[1]
    NCHUNK = E // CH
    ROWS_PER_TILE = N // NS
    mesh = plsc.VectorSubcoreMesh(core_axis_name="c", subcore_axis_name="s",
                                  num_cores=NC, num_subcores=NS)

    @functools.partial(
        pl.kernel,
        out_type=[
            jax.ShapeDtypeStruct((NC, N, DH), jnp.float32),
            jax.ShapeDtypeStruct((N, 16), jnp.float32),
        ],
        mesh=mesh,
        scratch_types=[
            pltpu.VMEM((CH,), jnp.int32), pltpu.VMEM((CH,), jnp.int32),
            pltpu.VMEM((CH,), jnp.int32), pltpu.VMEM((CH,), jnp.int32),
            pltpu.VMEM((CH, DH), jnp.float32), pltpu.VMEM((CH, DH), jnp.float32),
            pltpu.VMEM((CH, DH), jnp.float32), pltpu.VMEM((CH, DH), jnp.float32),
            pltpu.VMEM((CH, 16), jnp.float32), pltpu.VMEM((CH, 16), jnp.float32),
            pltpu.SemaphoreType.DMA, pltpu.SemaphoreType.DMA,
            pltpu.SemaphoreType.DMA, pltpu.SemaphoreType.DMA,
            pltpu.VMEM_SHARED((N, DH), jnp.float32),
            pltpu.VMEM_SHARED((N, 16), jnp.float32),
        ],
        compiler_params=pltpu.CompilerParams(use_tc_tiling_on_sc=False),
    )
    def k(ei, hl_hbm, hr_hbm, ml_hbm, mr_hbm, pv_hbm, hp_o, pp_o,
          row0, col0, row1, col1, hb0, hb1, mb0, mb1, pb0, pb1,
          g0, g1, w0, w1, hsh, psh):
        c = lax.axis_index("c")
        s = lax.axis_index("s")
        n_mine = (NCHUNK - s + NS - 1) // NS
        slots = ((row0, col0, hb0, mb0, pb0, g0, w0),
                 (row1, col1, hb1, mb1, pb1, g1, w1))

        @plsc.parallel_loop(0, CH, unroll=4)
        def zero_bufs(i):
            for j in range(0, DH, 16):
                hb0[i, pl.ds(j, 16)] = jnp.zeros((16,), jnp.float32)
            pb0[i, pl.ds(0, 16)] = jnp.zeros((16,), jnp.float32)

        r0 = s * ROWS_PER_TILE
        full = ROWS_PER_TILE // CH
        tail = ROWS_PER_TILE - full * CH

        def zero_sh(q, carry):
            pltpu.sync_copy(hb0, hsh.at[pl.ds(r0 + q * CH, CH)])
            pltpu.sync_copy(pb0, psh.at[pl.ds(r0 + q * CH, CH)])
            return carry

        lax.fori_loop(0, full, zero_sh, 0)

        @pl.when(tail > 0)
        def _():
            pltpu.sync_copy(hb0.at[pl.ds(0, tail)],
                            hsh.at[pl.ds(r0 + full * CH, tail)])
            pltpu.sync_copy(pb0.at[pl.ds(0, tail)],
                            psh.at[pl.ds(r0 + full * CH, tail)])

        plsc.subcore_barrier()

        def chunk_base(i):
            return (s + i * NS) * CH

        def issue(i, slot):
            rowv, colv, hbuf, mbuf, pbuf, g, w = slot
            base = chunk_base(i)
            pltpu.sync_copy(ei.at[0, pl.ds(base, CH)], rowv)
            pltpu.sync_copy(ei.at[1, pl.ds(base, CH)], colv)

            @pl.when(c == 0)
            def _():
                pltpu.async_copy(hl_hbm.at[rowv], hbuf, g)
                pltpu.async_copy(ml_hbm.at[pl.ds(base, CH)], mbuf, g)
                pltpu.async_copy(pv_hbm.at[pl.ds(base, CH)], pbuf, g)

            @pl.when(c == 1)
            def _():
                pltpu.async_copy(hr_hbm.at[rowv], hbuf, g)
                pltpu.async_copy(mr_hbm.at[pl.ds(base, CH)], mbuf, g)

        def wait_gathers(i, slot):
            rowv, colv, hbuf, mbuf, pbuf, g, w = slot
            base = chunk_base(i)
            pltpu.make_async_copy(hl_hbm.at[rowv], hbuf, g).wait()
            pltpu.make_async_copy(ml_hbm.at[pl.ds(base, CH)], mbuf, g).wait()

            @pl.when(c == 0)
            def _():
                pltpu.make_async_copy(pv_hbm.at[pl.ds(base, CH)], pbuf, g).wait()

        def compute_and_scatter(slot):
            rowv, colv, hbuf, mbuf, pbuf, g, w = slot

            @plsc.parallel_loop(0, CH, unroll=4)
            def mul(ii):
                for jj in range(0, DH, 16):
                    hbuf[ii, pl.ds(jj, 16)] = (
                        hbuf[ii, pl.ds(jj, 16)] * mbuf[ii, pl.ds(jj, 16)]
                    )
            pltpu.async_copy(hbuf, hsh.at[colv], w, add=True)

            @pl.when(c == 0)
            def _():
                pltpu.async_copy(pbuf, psh.at[colv], w, add=True)

        def wait_scatter(slot):
            rowv, colv, hbuf, mbuf, pbuf, g, w = slot
            pltpu.make_async_copy(hbuf, hsh.at[colv], w).wait()

            @pl.when(c == 0)
            def _():
                pltpu.make_async_copy(pbuf, psh.at[colv], w).wait()

        @pl.when(n_mine > 0)
        def _():
            issue(0, slots[0])

        def body(i2, carry):
            a = 2 * i2
            b = a + 1

            @pl.when(b < n_mine)
            def _():
                @pl.when(i2 > 0)
                def _():
                    wait_scatter(slots[1])
                issue(b, slots[1])

            wait_gathers(a, slots[0])
            compute_and_scatter(slots[0])

            @pl.when(b < n_mine)
            def _():
                wait_gathers(b, slots[1])
                compute_and_scatter(slots[1])

            @pl.when(a + 2 < n_mine)
            def _():
                wait_scatter(slots[0])
                issue(a + 2, slots[0])

            return carry

        lax.fori_loop(0, (n_mine + 1) // 2, body, 0)

        @pl.when(n_mine > 0)
        def _():
            wait_scatter(slots[0])

        @pl.when(n_mine > 1)
        def _():
            wait_scatter(slots[1])

        plsc.subcore_barrier()
        pltpu.sync_copy(hsh.at[pl.ds(r0, ROWS_PER_TILE)],
                        hp_o.at[c, pl.ds(r0, ROWS_PER_TILE)])

        @pl.when(c == 0)
        def _():
            pltpu.sync_copy(psh.at[pl.ds(r0, ROWS_PER_TILE)],
                            pp_o.at[pl.ds(r0, ROWS_PER_TILE)])

    return k(edge_index, hL, hR, mL, mR, posv)


def _assemble_h(hp):
    _, N, DH = hp.shape
    BLK = 2000

    def body(hp_ref, out_ref):
        out_ref[:, 0:DH] = hp_ref[0]
        out_ref[:, DH:2 * DH] = hp_ref[1]

    return pl.pallas_call(
        body,
        grid=(N // BLK,),
        in_specs=[pl.BlockSpec((NC, BLK, DH), lambda i: (0, i, 0))],
        out_specs=pl.BlockSpec((BLK, 2 * DH), lambda i: (i, 0)),
        out_shape=jax.ShapeDtypeStruct((N, 2 * DH), jnp.float32),
    )(hp)


def _expand_pos(pp, num_out):
    N, _ = pp.shape

    def body(pp_ref, out_ref):
        i = pl.program_id(0)
        val = pp_ref[:, 0:3]
        out_ref[...] = jnp.where(i == 0, val, jnp.zeros_like(val))

    return pl.pallas_call(
        body,
        grid=(num_out // N,),
        in_specs=[pl.BlockSpec((N, 16), lambda i: (0, 0))],
        out_specs=pl.BlockSpec((N, 3), lambda i: (i, 0)),
        out_shape=jax.ShapeDtypeStruct((num_out, 3), jnp.float32),
    )(pp)


def kernel(h, pos, edge_attr, W_e1, b_e1, W_e2, b_e2,
           W_x1, b_x1, W_x2, b_x2, edge_index):
    del edge_attr
    N, D = h.shape
    E = edge_index.shape[1]
    num_out = max(N, E)

    pos16 = jnp.pad(pos, ((0, 0), (0, 13)))
    wd_row = W_e1[2 * D].reshape(1, D)
    hL = h[:, 0:64]
    hR = h[:, 64:128]

    A, B = _precompute_ab(h, W_e1, b_e1.reshape(1, D))
    presum, rel16 = _sc_gather(edge_index, A, B, pos16)
    mL, mR, posv = _tc_edge_mlp(presum, rel16, wd_row, W_e2,
                                b_e2.reshape(1, D), W_x1, b_x1.reshape(1, D),
                                W_x2, b_x2.reshape(1, 1))
    hp, pp = _sc_scatter(edge_index, hL, hR, mL, mR, posv)
    h_msg = _assemble_h(hp)
    pos_msg = _expand_pos(pp, num_out)
    return (h_msg, pos_msg)
